# baseline (speedup 1.0000x reference)
import functools
import math

import jax
import jax.numpy as jnp
from jax import lax
from jax.experimental import pallas as pl
from jax.experimental.pallas import tpu as pltpu
from jax.experimental.pallas import tpu_sc as plsc

VOCAB = 1000000
D = 64
T = 200
SCALE = math.sqrt(D)
B = 4096

NC = 2
NS = 16
NW = NC * NS
SEQ_PER_W = (B + NW - 1) // NW
LANES = 16
VPR = D // LANES
IDXW = 100


def _pos_encoding():
    position = jnp.arange(0, T, dtype=jnp.float32)[:, None]
    div_term = jnp.exp(
        jnp.arange(0, D, 2, dtype=jnp.float32) * (-(math.log(10000.0) / D)))
    pe = jnp.zeros((T, D), dtype=jnp.float32)
    pe = pe.at[:, 0::2].set(jnp.sin(position * div_term))
    pe = pe.at[:, 1::2].set(jnp.cos(position * div_term))
    return pe


@functools.partial(
    pl.kernel,
    mesh=plsc.VectorSubcoreMesh(core_axis_name="c", subcore_axis_name="s"),
    out_type=jax.ShapeDtypeStruct((B * T, D), jnp.float32),
    scratch_types=[
        pltpu.VMEM((T, D), jnp.float32),
        pltpu.VMEM((T // IDXW, IDXW), jnp.int32),
        pltpu.VMEM((T, D), jnp.float32),
        pltpu.SemaphoreType.DMA,
    ],
    compiler_params=pltpu.CompilerParams(use_tc_tiling_on_sc=False),
)
def _emb_kernel(x_hbm, tab_hbm, pe_hbm, out_hbm, pe_v, idx_v, rows_v, sem):
    wid = lax.axis_index("s") * NC + lax.axis_index("c")
    pltpu.sync_copy(pe_hbm, pe_v)

    def seq_body(i, _):
        seq = wid * SEQ_PER_W + i
        base = seq * T
        pltpu.sync_copy(x_hbm.at[pl.ds(seq * (T // IDXW), T // IDXW)], idx_v)
        for j in range(T // IDXW):
            pltpu.async_copy(
                tab_hbm.at[idx_v.at[j]],
                rows_v.at[pl.ds(j * IDXW, IDXW)],
                sem,
            ).wait()

        def row_body(r, _):
            for d in range(VPR):
                sl = pl.ds(d * LANES, LANES)
                rows_v[r, sl] = rows_v[r, sl] * SCALE + pe_v[r, sl]
            return ()

        lax.fori_loop(0, T, row_body, ())
        pltpu.sync_copy(rows_v, out_hbm.at[pl.ds(base, T)])
        return ()

    lax.fori_loop(0, SEQ_PER_W, seq_body, ())


def kernel(x, tok_emb):
    pe = _pos_encoding()
    x2 = x.reshape(B * T // IDXW, IDXW).astype(jnp.int32)
    out = _emb_kernel(x2, tok_emb, pe)
    return out.reshape(B, T, D)
